# Initial kernel scaffold; baseline (speedup 1.0000x reference)
import jax
import jax.numpy as jnp
from jax.experimental import pallas as pl


def kernel(x, edge_index, batch, W1, b1, W2, b2, Wo1, bo1, Wo2, bo2):
    raise NotImplementedError("write your pallas kernel here")


# speedup vs baseline: 12.0040x; 12.0040x over previous
import functools

import jax
import jax.numpy as jnp
from jax import lax
from jax.experimental import pallas as pl
from jax.experimental.pallas import tpu as pltpu
from jax.experimental.pallas import tpu_sc as plsc

N = 100000
E = 3200000
HID = 32
HHID = HID // 2
NC = 2
NS = 16
LANES = 16
CH = 128
NPAD = 102400
TROWS = NPAD // NS
ZROWS = 1600
RB = 2048
GRID = NPAD // RB

_mesh = plsc.VectorSubcoreMesh(core_axis_name="c", subcore_axis_name="s")
_sc_params = pltpu.CompilerParams(use_tc_tiling_on_sc=False)


@functools.partial(
    pl.kernel,
    out_type=jax.ShapeDtypeStruct((NC, NPAD, HHID), jnp.float32),
    mesh=_mesh,
    scratch_types=[
        pltpu.VMEM((CH,), jnp.int32),
        pltpu.VMEM((CH, HHID), jnp.float32),
        pltpu.VMEM((ZROWS, HHID), jnp.float32),
        pltpu.VMEM_SHARED((NPAD, HHID), jnp.float32),
    ],
    compiler_params=_sc_params,
)
def _deg_kernel(dst_hbm, out_hbm, idx_v, ones_v, zb_v, acc_sh):
    c = lax.axis_index("c")
    s = lax.axis_index("s")

    zf = jnp.zeros((LANES,), jnp.float32)
    of = jnp.ones((LANES,), jnp.float32)

    def ofill(i, _):
        ones_v[i, :] = of
        return 0
    lax.fori_loop(0, CH, ofill, 0)

    def zfill(i, _):
        zb_v[i, :] = zf
        return 0
    lax.fori_loop(0, ZROWS, zfill, 0)
    row0 = s * TROWS
    for t in range(TROWS // ZROWS):
        pltpu.sync_copy(zb_v, acc_sh.at[pl.ds(row0 + t * ZROWS, ZROWS)])
    plsc.subcore_barrier()

    nch_sc = (E // NC) // CH
    q, r = nch_sc // NS, nch_sc % NS
    cnt = q + jnp.where(s < r, 1, 0)
    start = c * nch_sc + s * q + jnp.minimum(s, r)

    def body(j, _):
        off = pl.multiple_of((start + j) * CH, 8)
        pltpu.sync_copy(dst_hbm.at[pl.ds(off, CH)], idx_v)
        pltpu.sync_copy(ones_v, acc_sh.at[idx_v], add=True)
        return 0
    lax.fori_loop(0, cnt, body, 0)

    plsc.subcore_barrier()
    for t in range(TROWS // ZROWS):
        pltpu.sync_copy(acc_sh.at[pl.ds(row0 + t * ZROWS, ZROWS)], zb_v)
        pltpu.sync_copy(zb_v, out_hbm.at[c, pl.ds(row0 + t * ZROWS, ZROWS)])


@functools.partial(
    pl.kernel,
    out_type=jax.ShapeDtypeStruct((NC, NPAD, HHID), jnp.float32),
    mesh=_mesh,
    scratch_types=[
        pltpu.VMEM((CH,), jnp.int32),
        pltpu.VMEM((CH,), jnp.int32),
        pltpu.VMEM((CH, HHID), jnp.float32),
        pltpu.VMEM((ZROWS, HHID), jnp.float32),
        pltpu.VMEM_SHARED((NPAD, HHID), jnp.float32),
    ],
    compiler_params=_sc_params,
)
def _edge_kernel(g_hbm, src2_hbm, dst_hbm, out_hbm, sidx_v, didx_v, rows_v,
                 zb_v, acc_sh):
    c = lax.axis_index("c")
    s = lax.axis_index("s")

    zf = jnp.zeros((LANES,), jnp.float32)

    def zfill(i, _):
        zb_v[i, :] = zf
        return 0
    lax.fori_loop(0, ZROWS, zfill, 0)
    row0 = s * TROWS
    for t in range(TROWS // ZROWS):
        pltpu.sync_copy(zb_v, acc_sh.at[pl.ds(row0 + t * ZROWS, ZROWS)])
    plsc.subcore_barrier()

    nch = E // CH
    q, r = nch // NS, nch % NS
    cnt = q + jnp.where(s < r, 1, 0)
    start = s * q + jnp.minimum(s, r)
    cE = c * E

    def body(j, _):
        off = pl.multiple_of((start + j) * CH, 8)
        pltpu.sync_copy(src2_hbm.at[pl.ds(cE + off, CH)], sidx_v)
        pltpu.sync_copy(dst_hbm.at[pl.ds(off, CH)], didx_v)
        pltpu.sync_copy(g_hbm.at[sidx_v], rows_v)
        pltpu.sync_copy(rows_v, acc_sh.at[didx_v], add=True)
        return 0
    lax.fori_loop(0, cnt, body, 0)

    plsc.subcore_barrier()
    for t in range(TROWS // ZROWS):
        pltpu.sync_copy(acc_sh.at[pl.ds(row0 + t * ZROWS, ZROWS)], zb_v)
        pltpu.sync_copy(zb_v, out_hbm.at[c, pl.ds(row0 + t * ZROWS, ZROWS)])


def _dense1_body(x_ref, dp_ref, w1_ref, dinv_ref, h1_ref, g_ref):
    deg = dp_ref[0][:, :1] + dp_ref[1][:, :1] + 1.0
    dinv = lax.rsqrt(deg)
    h1 = jnp.dot(x_ref[...], w1_ref[...], preferred_element_type=jnp.float32)
    g = dinv * h1
    dinv_ref[...] = dinv
    h1_ref[...] = h1
    g_ref[...] = jnp.stack([g[:, :HHID], g[:, HHID:]], axis=0)


def _dense1(x, dp, W1):
    return pl.pallas_call(
        _dense1_body,
        grid=(GRID,),
        in_specs=[
            pl.BlockSpec((RB, 11), lambda i: (i, 0)),
            pl.BlockSpec((NC, RB, HHID), lambda i: (0, i, 0)),
            pl.BlockSpec((11, HID), lambda i: (0, 0)),
        ],
        out_specs=[
            pl.BlockSpec((RB, 1), lambda i: (i, 0)),
            pl.BlockSpec((RB, HID), lambda i: (i, 0)),
            pl.BlockSpec((NC, RB, HHID), lambda i: (0, i, 0)),
        ],
        out_shape=[
            jax.ShapeDtypeStruct((NPAD, 1), jnp.float32),
            jax.ShapeDtypeStruct((NPAD, HID), jnp.float32),
            jax.ShapeDtypeStruct((NC, NPAD, HHID), jnp.float32),
        ],
    )(x, dp, W1)


def _dense2_body(acc_ref, dinv_ref, h1_ref, w2_ref, b1_ref, h2_ref, g_ref):
    dinv = dinv_ref[...]
    agg = jnp.concatenate([acc_ref[0], acc_ref[1]], axis=-1)
    pre = dinv * agg + (dinv * dinv) * h1_ref[...] + b1_ref[...]
    h = jnp.maximum(pre, 0.0)
    h2 = jnp.dot(h, w2_ref[...], preferred_element_type=jnp.float32)
    g2 = dinv * h2
    h2_ref[...] = h2
    g_ref[...] = jnp.stack([g2[:, :HHID], g2[:, HHID:]], axis=0)


def _dense2(acc, dinv, h1, W2, b1):
    return pl.pallas_call(
        _dense2_body,
        grid=(GRID,),
        in_specs=[
            pl.BlockSpec((NC, RB, HHID), lambda i: (0, i, 0)),
            pl.BlockSpec((RB, 1), lambda i: (i, 0)),
            pl.BlockSpec((RB, HID), lambda i: (i, 0)),
            pl.BlockSpec((HID, HID), lambda i: (0, 0)),
            pl.BlockSpec((1, HID), lambda i: (0, 0)),
        ],
        out_specs=[
            pl.BlockSpec((RB, HID), lambda i: (i, 0)),
            pl.BlockSpec((NC, RB, HHID), lambda i: (0, i, 0)),
        ],
        out_shape=[
            jax.ShapeDtypeStruct((NPAD, HID), jnp.float32),
            jax.ShapeDtypeStruct((NC, NPAD, HHID), jnp.float32),
        ],
    )(acc, dinv, h1, W2, b1)


def _dense3_body(acc_ref, dinv_ref, h2_ref, b2_ref, wo1_ref, bo1_ref,
                 wo2_ref, bo2_ref, y_ref):
    dinv = dinv_ref[...]
    agg = jnp.concatenate([acc_ref[0], acc_ref[1]], axis=-1)
    pre = dinv * agg + (dinv * dinv) * h2_ref[...] + b2_ref[...]
    h = jnp.maximum(pre, 0.0)
    t = jnp.dot(h, wo1_ref[...], preferred_element_type=jnp.float32)
    t = t + bo1_ref[...]
    t = jnp.where(t > 0, t, jnp.exp(t) - 1.0)
    y = jnp.dot(t, wo2_ref[...], preferred_element_type=jnp.float32)
    y_ref[...] = y + bo2_ref[...]


def _dense3(acc, dinv, h2, b2, Wo1, bo1, Wo2, bo2):
    return pl.pallas_call(
        _dense3_body,
        grid=(GRID,),
        in_specs=[
            pl.BlockSpec((NC, RB, HHID), lambda i: (0, i, 0)),
            pl.BlockSpec((RB, 1), lambda i: (i, 0)),
            pl.BlockSpec((RB, HID), lambda i: (i, 0)),
            pl.BlockSpec((1, HID), lambda i: (0, 0)),
            pl.BlockSpec((HID, HHID), lambda i: (0, 0)),
            pl.BlockSpec((1, HHID), lambda i: (0, 0)),
            pl.BlockSpec((HHID, 1), lambda i: (0, 0)),
            pl.BlockSpec((1, 1), lambda i: (0, 0)),
        ],
        out_specs=[pl.BlockSpec((RB, 1), lambda i: (i, 0))],
        out_shape=[jax.ShapeDtypeStruct((NPAD, 1), jnp.float32)],
    )(acc, dinv, h2, b2, Wo1, bo1, Wo2, bo2)


def kernel(x, edge_index, batch, W1, b1, W2, b2, Wo1, bo1, Wo2, bo2):
    del batch
    src = edge_index[0]
    dst = edge_index[1]
    src2 = jnp.concatenate([src, src + NPAD])
    x_pad = jnp.concatenate(
        [x, jnp.zeros((NPAD - N, x.shape[1]), x.dtype)], axis=0)

    dp = _deg_kernel(dst)
    dinv, h1, g1 = _dense1(x_pad, dp, W1)
    acc1 = _edge_kernel(g1.reshape(NC * NPAD, HHID), src2, dst)
    h2, g2 = _dense2(acc1, dinv, h1, W2, b1.reshape(1, HID))
    acc2 = _edge_kernel(g2.reshape(NC * NPAD, HHID), src2, dst)
    (y,) = _dense3(acc2, dinv, h2, b2.reshape(1, HID), Wo1,
                   bo1.reshape(1, HHID), Wo2, bo2.reshape(1, 1))
    return y[:N, 0]


# speedup vs baseline: 38.6229x; 3.2175x over previous
import functools

import jax
import jax.numpy as jnp
from jax import lax
from jax.experimental import pallas as pl
from jax.experimental.pallas import tpu as pltpu
from jax.experimental.pallas import tpu_sc as plsc

N = 100000
E = 3200000
HID = 32
HHID = HID // 2
NC = 2
NS = 16
LANES = 16
CH = 128
NPAD = 102400
TROWS = NPAD // NS
ZROWS = 400
RB = 2048
GRID = NPAD // RB

_mesh = plsc.VectorSubcoreMesh(core_axis_name="c", subcore_axis_name="s")
_sc_params = pltpu.CompilerParams(use_tc_tiling_on_sc=False)


@functools.partial(
    pl.kernel,
    out_type=jax.ShapeDtypeStruct((NC, NPAD, HHID), jnp.float32),
    mesh=_mesh,
    scratch_types=[
        pltpu.VMEM((CH,), jnp.int32),
        pltpu.VMEM((CH, HHID), jnp.float32),
        pltpu.VMEM((ZROWS, HHID), jnp.float32),
        pltpu.VMEM_SHARED((NPAD, HHID), jnp.float32),
    ],
    compiler_params=_sc_params,
)
def _deg_kernel(dst_hbm, out_hbm, idx_v, ones_v, zb_v, acc_sh):
    c = lax.axis_index("c")
    s = lax.axis_index("s")

    zf = jnp.zeros((LANES,), jnp.float32)
    of = jnp.ones((LANES,), jnp.float32)

    def ofill(i, _):
        ones_v[i, :] = of
        return 0
    lax.fori_loop(0, CH, ofill, 0)

    def zfill(i, _):
        zb_v[i, :] = zf
        return 0
    lax.fori_loop(0, ZROWS, zfill, 0)
    row0 = s * TROWS
    for t in range(TROWS // ZROWS):
        pltpu.sync_copy(zb_v, acc_sh.at[pl.ds(row0 + t * ZROWS, ZROWS)])
    plsc.subcore_barrier()

    nch_sc = (E // NC) // CH
    q, r = nch_sc // NS, nch_sc % NS
    cnt = q + jnp.where(s < r, 1, 0)
    start = c * nch_sc + s * q + jnp.minimum(s, r)

    def body(j, _):
        off = pl.multiple_of((start + j) * CH, 8)
        pltpu.sync_copy(dst_hbm.at[pl.ds(off, CH)], idx_v)
        pltpu.sync_copy(ones_v, acc_sh.at[idx_v], add=True)
        return 0
    lax.fori_loop(0, cnt, body, 0)

    plsc.subcore_barrier()
    for t in range(TROWS // ZROWS):
        pltpu.sync_copy(acc_sh.at[pl.ds(row0 + t * ZROWS, ZROWS)], zb_v)
        pltpu.sync_copy(zb_v, out_hbm.at[c, pl.ds(row0 + t * ZROWS, ZROWS)])


K = 4
BLK = K * CH
NBLK = E // BLK


@functools.partial(
    pl.kernel,
    out_type=jax.ShapeDtypeStruct((NC, NPAD, HHID), jnp.float32),
    mesh=_mesh,
    scratch_types=[
        pltpu.VMEM((2, BLK), jnp.int32),
        pltpu.VMEM((2, K, CH), jnp.int32),
        pltpu.VMEM((2, BLK, HHID), jnp.float32),
        pltpu.VMEM((ZROWS, HHID), jnp.float32),
        pltpu.VMEM_SHARED((NPAD, HHID), jnp.float32),
        pltpu.SemaphoreType.DMA,
        pltpu.SemaphoreType.DMA,
        pltpu.SemaphoreType.DMA,
        pltpu.SemaphoreType.DMA,
        pltpu.SemaphoreType.DMA,
        pltpu.SemaphoreType.DMA,
        pltpu.SemaphoreType.DMA,
        pltpu.SemaphoreType.DMA,
    ],
    compiler_params=_sc_params,
)
def _edge_kernel(g_hbm, src2_hbm, dstr_hbm, out_hbm, sbuf, dbuf, rows,
                 zb_v, acc_sh, si0, si1, sj0, sj1, sg0, sg1, ss0, ss1):
    c = lax.axis_index("c")
    s = lax.axis_index("s")
    sem_i = (si0, si1)
    sem_j = (sj0, sj1)
    sem_g = (sg0, sg1)
    sem_s = (ss0, ss1)

    zf = jnp.zeros((LANES,), jnp.float32)

    def zfill(i, _):
        zb_v[i, :] = zf
        return 0
    lax.fori_loop(0, ZROWS, zfill, 0)
    row0 = s * TROWS
    for t in range(TROWS // ZROWS):
        pltpu.sync_copy(zb_v, acc_sh.at[pl.ds(row0 + t * ZROWS, ZROWS)])
    plsc.subcore_barrier()

    q, r = NBLK // NS, NBLK % NS
    cnt = q + jnp.where(s < r, 1, 0)
    bstart = s * q + jnp.minimum(s, r)
    cE = c * E

    def idx_descs(b, u):
        off = pl.multiple_of((bstart + b) * BLK, 8)
        d_i = pltpu.make_async_copy(
            src2_hbm.at[pl.ds(cE + off, BLK)], sbuf.at[u], sem_i[u])
        d_j = pltpu.make_async_copy(
            dstr_hbm.at[pl.ds((bstart + b) * K, K)], dbuf.at[u], sem_j[u])
        return d_i, d_j

    def gather_desc(u, k):
        return pltpu.make_async_copy(
            g_hbm.at[sbuf.at[u, pl.ds(k * CH, CH)]],
            rows.at[u, pl.ds(k * CH, CH)], sem_g[u])

    def scatter_desc(u, k):
        return pltpu.make_async_copy(
            rows.at[u, pl.ds(k * CH, CH)], acc_sh.at[dbuf.at[u, k]],
            sem_s[u])

    def stage(b, u):
        w = b - 1

        @pl.when((w >= 0) & (w < cnt))
        def _():
            d_i, d_j = idx_descs(w, 1 - u)
            d_i.wait()
            d_j.wait()
            for k in range(K):
                gather_desc(1 - u, k).start()

        v = b - 2

        @pl.when((v >= 0) & (v < cnt))
        def _():
            for k in range(K):
                gather_desc(u, k).wait()
            for k in range(K):
                pltpu.async_copy(rows.at[u, pl.ds(k * CH, CH)],
                                 acc_sh.at[dbuf.at[u, k]], sem_s[u],
                                 add=True)
            for k in range(K):
                scatter_desc(u, k).wait()

        @pl.when(b < cnt)
        def _():
            d_i, d_j = idx_descs(b, u)
            d_i.start()
            d_j.start()

    def body(js, _):
        stage(2 * js, 0)
        stage(2 * js + 1, 1)
        return 0
    lax.fori_loop(0, (cnt + 3) // 2, body, 0)

    plsc.subcore_barrier()
    for t in range(TROWS // ZROWS):
        pltpu.sync_copy(acc_sh.at[pl.ds(row0 + t * ZROWS, ZROWS)], zb_v)
        pltpu.sync_copy(zb_v, out_hbm.at[c, pl.ds(row0 + t * ZROWS, ZROWS)])


def _dense1_body(x_ref, dp_ref, w1_ref, dinv_ref, h1_ref, g_ref):
    deg = dp_ref[0][:, :1] + dp_ref[1][:, :1] + 1.0
    dinv = lax.rsqrt(deg)
    h1 = jnp.dot(x_ref[...], w1_ref[...], preferred_element_type=jnp.float32)
    g = dinv * h1
    dinv_ref[...] = dinv
    h1_ref[...] = h1
    g_ref[...] = jnp.stack([g[:, :HHID], g[:, HHID:]], axis=0)


def _dense1(x, dp, W1):
    return pl.pallas_call(
        _dense1_body,
        grid=(GRID,),
        in_specs=[
            pl.BlockSpec((RB, 11), lambda i: (i, 0)),
            pl.BlockSpec((NC, RB, HHID), lambda i: (0, i, 0)),
            pl.BlockSpec((11, HID), lambda i: (0, 0)),
        ],
        out_specs=[
            pl.BlockSpec((RB, 1), lambda i: (i, 0)),
            pl.BlockSpec((RB, HID), lambda i: (i, 0)),
            pl.BlockSpec((NC, RB, HHID), lambda i: (0, i, 0)),
        ],
        out_shape=[
            jax.ShapeDtypeStruct((NPAD, 1), jnp.float32),
            jax.ShapeDtypeStruct((NPAD, HID), jnp.float32),
            jax.ShapeDtypeStruct((NC, NPAD, HHID), jnp.float32),
        ],
    )(x, dp, W1)


def _dense2_body(acc_ref, dinv_ref, h1_ref, w2_ref, b1_ref, h2_ref, g_ref):
    dinv = dinv_ref[...]
    agg = jnp.concatenate([acc_ref[0], acc_ref[1]], axis=-1)
    pre = dinv * agg + (dinv * dinv) * h1_ref[...] + b1_ref[...]
    h = jnp.maximum(pre, 0.0)
    h2 = jnp.dot(h, w2_ref[...], preferred_element_type=jnp.float32)
    g2 = dinv * h2
    h2_ref[...] = h2
    g_ref[...] = jnp.stack([g2[:, :HHID], g2[:, HHID:]], axis=0)


def _dense2(acc, dinv, h1, W2, b1):
    return pl.pallas_call(
        _dense2_body,
        grid=(GRID,),
        in_specs=[
            pl.BlockSpec((NC, RB, HHID), lambda i: (0, i, 0)),
            pl.BlockSpec((RB, 1), lambda i: (i, 0)),
            pl.BlockSpec((RB, HID), lambda i: (i, 0)),
            pl.BlockSpec((HID, HID), lambda i: (0, 0)),
            pl.BlockSpec((1, HID), lambda i: (0, 0)),
        ],
        out_specs=[
            pl.BlockSpec((RB, HID), lambda i: (i, 0)),
            pl.BlockSpec((NC, RB, HHID), lambda i: (0, i, 0)),
        ],
        out_shape=[
            jax.ShapeDtypeStruct((NPAD, HID), jnp.float32),
            jax.ShapeDtypeStruct((NC, NPAD, HHID), jnp.float32),
        ],
    )(acc, dinv, h1, W2, b1)


def _dense3_body(acc_ref, dinv_ref, h2_ref, b2_ref, wo1_ref, bo1_ref,
                 wo2_ref, bo2_ref, y_ref):
    dinv = dinv_ref[...]
    agg = jnp.concatenate([acc_ref[0], acc_ref[1]], axis=-1)
    pre = dinv * agg + (dinv * dinv) * h2_ref[...] + b2_ref[...]
    h = jnp.maximum(pre, 0.0)
    t = jnp.dot(h, wo1_ref[...], preferred_element_type=jnp.float32)
    t = t + bo1_ref[...]
    t = jnp.where(t > 0, t, jnp.exp(t) - 1.0)
    y = jnp.dot(t, wo2_ref[...], preferred_element_type=jnp.float32)
    y_ref[...] = y + bo2_ref[...]


def _dense3(acc, dinv, h2, b2, Wo1, bo1, Wo2, bo2):
    return pl.pallas_call(
        _dense3_body,
        grid=(GRID,),
        in_specs=[
            pl.BlockSpec((NC, RB, HHID), lambda i: (0, i, 0)),
            pl.BlockSpec((RB, 1), lambda i: (i, 0)),
            pl.BlockSpec((RB, HID), lambda i: (i, 0)),
            pl.BlockSpec((1, HID), lambda i: (0, 0)),
            pl.BlockSpec((HID, HHID), lambda i: (0, 0)),
            pl.BlockSpec((1, HHID), lambda i: (0, 0)),
            pl.BlockSpec((HHID, 1), lambda i: (0, 0)),
            pl.BlockSpec((1, 1), lambda i: (0, 0)),
        ],
        out_specs=[pl.BlockSpec((RB, 1), lambda i: (i, 0))],
        out_shape=[jax.ShapeDtypeStruct((NPAD, 1), jnp.float32)],
    )(acc, dinv, h2, b2, Wo1, bo1, Wo2, bo2)


def kernel(x, edge_index, batch, W1, b1, W2, b2, Wo1, bo1, Wo2, bo2):
    del batch
    src = edge_index[0]
    dst = edge_index[1]
    src2 = jnp.concatenate([src, src + NPAD])
    dstr = dst.reshape(E // CH, CH)
    x_pad = jnp.concatenate(
        [x, jnp.zeros((NPAD - N, x.shape[1]), x.dtype)], axis=0)

    dp = _deg_kernel(dst)
    dinv, h1, g1 = _dense1(x_pad, dp, W1)
    acc1 = _edge_kernel(g1.reshape(NC * NPAD, HHID), src2, dstr)
    h2, g2 = _dense2(acc1, dinv, h1, W2, b1.reshape(1, HID))
    acc2 = _edge_kernel(g2.reshape(NC * NPAD, HHID), src2, dstr)
    (y,) = _dense3(acc2, dinv, h2, b2.reshape(1, HID), Wo1,
                   bo1.reshape(1, HHID), Wo2, bo2.reshape(1, 1))
    return y[:N, 0]


# speedup vs baseline: 49.0434x; 1.2698x over previous
import functools

import jax
import jax.numpy as jnp
from jax import lax
from jax.experimental import pallas as pl
from jax.experimental.pallas import tpu as pltpu
from jax.experimental.pallas import tpu_sc as plsc

N = 100000
E = 3200000
HID = 32
HHID = HID // 2
NC = 2
NS = 16
LANES = 16
CH = 128
NPAD = 102400
TROWS = NPAD // NS
ZROWS = 400
RB = 2048
GRID = NPAD // RB

_mesh = plsc.VectorSubcoreMesh(core_axis_name="c", subcore_axis_name="s")
_sc_params = pltpu.CompilerParams(use_tc_tiling_on_sc=False)


K = 4
BLK = K * CH


@functools.partial(
    pl.kernel,
    out_type=jax.ShapeDtypeStruct((NC, NPAD), jnp.float32),
    mesh=_mesh,
    scratch_types=[
        pltpu.VMEM((2, K, CH), jnp.int32),
        pltpu.VMEM((CH,), jnp.float32),
        pltpu.VMEM((TROWS,), jnp.float32),
        pltpu.VMEM_SHARED((NPAD,), jnp.float32),
        pltpu.SemaphoreType.DMA,
        pltpu.SemaphoreType.DMA,
        pltpu.SemaphoreType.DMA,
        pltpu.SemaphoreType.DMA,
    ],
    compiler_params=_sc_params,
)
def _deg_kernel(dstr_hbm, out_hbm, dbuf, ones_v, zb_v, acc_sh,
                sj0, sj1, ss0, ss1):
    c = lax.axis_index("c")
    s = lax.axis_index("s")
    sem_j = (sj0, sj1)
    sem_s = (ss0, ss1)

    zf = jnp.zeros((LANES,), jnp.float32)
    of = jnp.ones((LANES,), jnp.float32)
    for k in range(CH // LANES):
        ones_v[pl.ds(k * LANES, LANES)] = of

    def zfill(i, _):
        zb_v[pl.ds(i * LANES, LANES)] = zf
        return 0
    lax.fori_loop(0, TROWS // LANES, zfill, 0)
    row0 = s * TROWS
    pltpu.sync_copy(zb_v, acc_sh.at[pl.ds(row0, TROWS)])
    plsc.subcore_barrier()

    nblk_sc = (E // NC) // BLK
    q, r = nblk_sc // NS, nblk_sc % NS
    cnt = q + jnp.where(s < r, 1, 0)
    bstart = s * q + jnp.minimum(s, r)
    crow = c * ((E // NC) // CH)

    def idx_desc(b, u):
        return pltpu.make_async_copy(
            dstr_hbm.at[pl.ds(crow + (bstart + b) * K, K)], dbuf.at[u],
            sem_j[u])

    def scatter_desc(u, k):
        return pltpu.make_async_copy(ones_v, acc_sh.at[dbuf.at[u, k]],
                                     sem_s[u])

    def stage(b, u):
        w = b - 1

        @pl.when((w >= 0) & (w < cnt))
        def _():
            idx_desc(w, 1 - u).wait()
            for k in range(K):
                pltpu.async_copy(ones_v, acc_sh.at[dbuf.at[1 - u, k]],
                                 sem_s[1 - u], add=True)

        v = b - 2

        @pl.when((v >= 0) & (v < cnt))
        def _():
            for k in range(K):
                scatter_desc(u, k).wait()

        @pl.when(b < cnt)
        def _():
            idx_desc(b, u).start()

    def body(js, _):
        stage(2 * js, 0)
        stage(2 * js + 1, 1)
        return 0
    lax.fori_loop(0, (cnt + 3) // 2, body, 0)

    plsc.subcore_barrier()
    pltpu.sync_copy(acc_sh.at[pl.ds(row0, TROWS)], zb_v)
    pltpu.sync_copy(zb_v, out_hbm.at[c, pl.ds(row0, TROWS)])


NBLK = E // BLK


@functools.partial(
    pl.kernel,
    out_type=jax.ShapeDtypeStruct((NC, NPAD, HHID), jnp.float32),
    mesh=_mesh,
    scratch_types=[
        pltpu.VMEM((2, BLK), jnp.int32),
        pltpu.VMEM((2, K, CH), jnp.int32),
        pltpu.VMEM((2, BLK, HHID), jnp.float32),
        pltpu.VMEM((ZROWS, HHID), jnp.float32),
        pltpu.VMEM_SHARED((NPAD, HHID), jnp.float32),
        pltpu.SemaphoreType.DMA,
        pltpu.SemaphoreType.DMA,
        pltpu.SemaphoreType.DMA,
        pltpu.SemaphoreType.DMA,
        pltpu.SemaphoreType.DMA,
        pltpu.SemaphoreType.DMA,
        pltpu.SemaphoreType.DMA,
        pltpu.SemaphoreType.DMA,
    ],
    compiler_params=_sc_params,
)
def _edge_kernel(g_hbm, src2_hbm, dstr_hbm, out_hbm, sbuf, dbuf, rows,
                 zb_v, acc_sh, si0, si1, sj0, sj1, sg0, sg1, ss0, ss1):
    c = lax.axis_index("c")
    s = lax.axis_index("s")
    sem_i = (si0, si1)
    sem_j = (sj0, sj1)
    sem_g = (sg0, sg1)
    sem_s = (ss0, ss1)

    zf = jnp.zeros((LANES,), jnp.float32)

    def zfill(i, _):
        zb_v[i, :] = zf
        return 0
    lax.fori_loop(0, ZROWS, zfill, 0)
    row0 = s * TROWS
    for t in range(TROWS // ZROWS):
        pltpu.sync_copy(zb_v, acc_sh.at[pl.ds(row0 + t * ZROWS, ZROWS)])
    plsc.subcore_barrier()

    q, r = NBLK // NS, NBLK % NS
    cnt = q + jnp.where(s < r, 1, 0)
    bstart = s * q + jnp.minimum(s, r)
    cE = c * E

    def idx_descs(b, u):
        off = pl.multiple_of((bstart + b) * BLK, 8)
        d_i = pltpu.make_async_copy(
            src2_hbm.at[pl.ds(cE + off, BLK)], sbuf.at[u], sem_i[u])
        d_j = pltpu.make_async_copy(
            dstr_hbm.at[pl.ds((bstart + b) * K, K)], dbuf.at[u], sem_j[u])
        return d_i, d_j

    def gather_desc(u, k):
        return pltpu.make_async_copy(
            g_hbm.at[sbuf.at[u, pl.ds(k * CH, CH)]],
            rows.at[u, pl.ds(k * CH, CH)], sem_g[u])

    def scatter_desc(u, k):
        return pltpu.make_async_copy(
            rows.at[u, pl.ds(k * CH, CH)], acc_sh.at[dbuf.at[u, k]],
            sem_s[u])

    def stage(b, u):
        w = b - 1

        @pl.when((w >= 0) & (w < cnt))
        def _():
            d_i, d_j = idx_descs(w, 1 - u)
            d_i.wait()
            d_j.wait()
            for k in range(K):
                gather_desc(1 - u, k).start()

        v = b - 2

        @pl.when((v >= 0) & (v < cnt))
        def _():
            for k in range(K):
                gather_desc(u, k).wait()
            for k in range(K):
                pltpu.async_copy(rows.at[u, pl.ds(k * CH, CH)],
                                 acc_sh.at[dbuf.at[u, k]], sem_s[u],
                                 add=True)
            for k in range(K):
                scatter_desc(u, k).wait()

        @pl.when(b < cnt)
        def _():
            d_i, d_j = idx_descs(b, u)
            d_i.start()
            d_j.start()

    def body(js, _):
        stage(2 * js, 0)
        stage(2 * js + 1, 1)
        return 0
    lax.fori_loop(0, (cnt + 3) // 2, body, 0)

    plsc.subcore_barrier()
    for t in range(TROWS // ZROWS):
        pltpu.sync_copy(acc_sh.at[pl.ds(row0 + t * ZROWS, ZROWS)], zb_v)
        pltpu.sync_copy(zb_v, out_hbm.at[c, pl.ds(row0 + t * ZROWS, ZROWS)])


def _dense1_body(x_ref, dp_ref, w1_ref, dinv_ref, h1_ref, g_ref):
    deg = dp_ref[0] + dp_ref[1] + 1.0
    dinv = lax.rsqrt(deg)[:, None]
    h1 = jnp.dot(x_ref[...], w1_ref[...], preferred_element_type=jnp.float32)
    g = dinv * h1
    dinv_ref[...] = dinv
    h1_ref[...] = h1
    g_ref[...] = jnp.stack([g[:, :HHID], g[:, HHID:]], axis=0)


def _dense1(x, dp, W1):
    return pl.pallas_call(
        _dense1_body,
        grid=(GRID,),
        in_specs=[
            pl.BlockSpec((RB, 11), lambda i: (i, 0)),
            pl.BlockSpec((NC, RB), lambda i: (0, i)),
            pl.BlockSpec((11, HID), lambda i: (0, 0)),
        ],
        out_specs=[
            pl.BlockSpec((RB, 1), lambda i: (i, 0)),
            pl.BlockSpec((RB, HID), lambda i: (i, 0)),
            pl.BlockSpec((NC, RB, HHID), lambda i: (0, i, 0)),
        ],
        out_shape=[
            jax.ShapeDtypeStruct((NPAD, 1), jnp.float32),
            jax.ShapeDtypeStruct((NPAD, HID), jnp.float32),
            jax.ShapeDtypeStruct((NC, NPAD, HHID), jnp.float32),
        ],
    )(x, dp, W1)


def _dense2_body(acc_ref, dinv_ref, h1_ref, w2_ref, b1_ref, h2_ref, g_ref):
    dinv = dinv_ref[...]
    agg = jnp.concatenate([acc_ref[0], acc_ref[1]], axis=-1)
    pre = dinv * agg + (dinv * dinv) * h1_ref[...] + b1_ref[...]
    h = jnp.maximum(pre, 0.0)
    h2 = jnp.dot(h, w2_ref[...], preferred_element_type=jnp.float32)
    g2 = dinv * h2
    h2_ref[...] = h2
    g_ref[...] = jnp.stack([g2[:, :HHID], g2[:, HHID:]], axis=0)


def _dense2(acc, dinv, h1, W2, b1):
    return pl.pallas_call(
        _dense2_body,
        grid=(GRID,),
        in_specs=[
            pl.BlockSpec((NC, RB, HHID), lambda i: (0, i, 0)),
            pl.BlockSpec((RB, 1), lambda i: (i, 0)),
            pl.BlockSpec((RB, HID), lambda i: (i, 0)),
            pl.BlockSpec((HID, HID), lambda i: (0, 0)),
            pl.BlockSpec((1, HID), lambda i: (0, 0)),
        ],
        out_specs=[
            pl.BlockSpec((RB, HID), lambda i: (i, 0)),
            pl.BlockSpec((NC, RB, HHID), lambda i: (0, i, 0)),
        ],
        out_shape=[
            jax.ShapeDtypeStruct((NPAD, HID), jnp.float32),
            jax.ShapeDtypeStruct((NC, NPAD, HHID), jnp.float32),
        ],
    )(acc, dinv, h1, W2, b1)


def _dense3_body(acc_ref, dinv_ref, h2_ref, b2_ref, wo1_ref, bo1_ref,
                 wo2_ref, bo2_ref, y_ref):
    dinv = dinv_ref[...]
    agg = jnp.concatenate([acc_ref[0], acc_ref[1]], axis=-1)
    pre = dinv * agg + (dinv * dinv) * h2_ref[...] + b2_ref[...]
    h = jnp.maximum(pre, 0.0)
    t = jnp.dot(h, wo1_ref[...], preferred_element_type=jnp.float32)
    t = t + bo1_ref[...]
    t = jnp.where(t > 0, t, jnp.exp(t) - 1.0)
    y = jnp.dot(t, wo2_ref[...], preferred_element_type=jnp.float32)
    y_ref[...] = y + bo2_ref[...]


def _dense3(acc, dinv, h2, b2, Wo1, bo1, Wo2, bo2):
    return pl.pallas_call(
        _dense3_body,
        grid=(GRID,),
        in_specs=[
            pl.BlockSpec((NC, RB, HHID), lambda i: (0, i, 0)),
            pl.BlockSpec((RB, 1), lambda i: (i, 0)),
            pl.BlockSpec((RB, HID), lambda i: (i, 0)),
            pl.BlockSpec((1, HID), lambda i: (0, 0)),
            pl.BlockSpec((HID, HHID), lambda i: (0, 0)),
            pl.BlockSpec((1, HHID), lambda i: (0, 0)),
            pl.BlockSpec((HHID, 1), lambda i: (0, 0)),
            pl.BlockSpec((1, 1), lambda i: (0, 0)),
        ],
        out_specs=[pl.BlockSpec((RB, 1), lambda i: (i, 0))],
        out_shape=[jax.ShapeDtypeStruct((NPAD, 1), jnp.float32)],
    )(acc, dinv, h2, b2, Wo1, bo1, Wo2, bo2)


def kernel(x, edge_index, batch, W1, b1, W2, b2, Wo1, bo1, Wo2, bo2):
    del batch
    src = edge_index[0]
    dst = edge_index[1]
    src2 = jnp.concatenate([src, src + NPAD])
    dstr = dst.reshape(E // CH, CH)
    x_pad = jnp.concatenate(
        [x, jnp.zeros((NPAD - N, x.shape[1]), x.dtype)], axis=0)

    dp = _deg_kernel(dstr)
    dinv, h1, g1 = _dense1(x_pad, dp, W1)
    acc1 = _edge_kernel(g1.reshape(NC * NPAD, HHID), src2, dstr)
    h2, g2 = _dense2(acc1, dinv, h1, W2, b1.reshape(1, HID))
    acc2 = _edge_kernel(g2.reshape(NC * NPAD, HHID), src2, dstr)
    (y,) = _dense3(acc2, dinv, h2, b2.reshape(1, HID), Wo1,
                   bo1.reshape(1, HHID), Wo2, bo2.reshape(1, 1))
    return y[:N, 0]


# speedup vs baseline: 52.4282x; 1.0690x over previous
import functools

import jax
import jax.numpy as jnp
from jax import lax
from jax.experimental import pallas as pl
from jax.experimental.pallas import tpu as pltpu
from jax.experimental.pallas import tpu_sc as plsc

N = 100000
E = 3200000
HID = 32
HHID = HID // 2
NC = 2
NS = 16
LANES = 16
CH = 128
NPAD = 102400
TROWS = NPAD // NS
ZROWS = 320
RB = 2048
GRID = NPAD // RB

_mesh = plsc.VectorSubcoreMesh(core_axis_name="c", subcore_axis_name="s")
_sc_params = pltpu.CompilerParams(use_tc_tiling_on_sc=False)


K = 5
BLK = K * CH


@functools.partial(
    pl.kernel,
    out_type=jax.ShapeDtypeStruct((NC, NPAD), jnp.float32),
    mesh=_mesh,
    scratch_types=[
        pltpu.VMEM((2, K, CH), jnp.int32),
        pltpu.VMEM((CH,), jnp.float32),
        pltpu.VMEM((TROWS,), jnp.float32),
        pltpu.VMEM_SHARED((NPAD,), jnp.float32),
        pltpu.SemaphoreType.DMA,
        pltpu.SemaphoreType.DMA,
        pltpu.SemaphoreType.DMA,
        pltpu.SemaphoreType.DMA,
    ],
    compiler_params=_sc_params,
)
def _deg_kernel(dstr_hbm, out_hbm, dbuf, ones_v, zb_v, acc_sh,
                sj0, sj1, ss0, ss1):
    c = lax.axis_index("c")
    s = lax.axis_index("s")
    sem_j = (sj0, sj1)
    sem_s = (ss0, ss1)

    zf = jnp.zeros((LANES,), jnp.float32)
    of = jnp.ones((LANES,), jnp.float32)
    for k in range(CH // LANES):
        ones_v[pl.ds(k * LANES, LANES)] = of

    def zfill(i, _):
        zb_v[pl.ds(i * LANES, LANES)] = zf
        return 0
    lax.fori_loop(0, TROWS // LANES, zfill, 0)
    row0 = s * TROWS
    pltpu.sync_copy(zb_v, acc_sh.at[pl.ds(row0, TROWS)])
    plsc.subcore_barrier()

    nblk_sc = (E // NC) // BLK
    q, r = nblk_sc // NS, nblk_sc % NS
    cnt = q + jnp.where(s < r, 1, 0)
    bstart = s * q + jnp.minimum(s, r)
    crow = c * ((E // NC) // CH)

    def idx_desc(b, u):
        return pltpu.make_async_copy(
            dstr_hbm.at[pl.ds(crow + (bstart + b) * K, K)], dbuf.at[u],
            sem_j[u])

    def scatter_desc(u, k):
        return pltpu.make_async_copy(ones_v, acc_sh.at[dbuf.at[u, k]],
                                     sem_s[u])

    def stage(b, u):
        w = b - 1

        @pl.when((w >= 0) & (w < cnt))
        def _():
            idx_desc(w, 1 - u).wait()
            for k in range(K):
                pltpu.async_copy(ones_v, acc_sh.at[dbuf.at[1 - u, k]],
                                 sem_s[1 - u], add=True)

        v = b - 2

        @pl.when((v >= 0) & (v < cnt))
        def _():
            for k in range(K):
                scatter_desc(u, k).wait()

        @pl.when(b < cnt)
        def _():
            idx_desc(b, u).start()

    def body(js, _):
        stage(2 * js, 0)
        stage(2 * js + 1, 1)
        return 0
    lax.fori_loop(0, (cnt + 3) // 2, body, 0)

    plsc.subcore_barrier()
    pltpu.sync_copy(acc_sh.at[pl.ds(row0, TROWS)], zb_v)
    pltpu.sync_copy(zb_v, out_hbm.at[c, pl.ds(row0, TROWS)])


NBLK = E // BLK


@functools.partial(
    pl.kernel,
    out_type=jax.ShapeDtypeStruct((NC, NPAD, HHID), jnp.float32),
    mesh=_mesh,
    scratch_types=[
        pltpu.VMEM((2, BLK), jnp.int32),
        pltpu.VMEM((2, K, CH), jnp.int32),
        pltpu.VMEM((2, BLK, HHID), jnp.float32),
        pltpu.VMEM((ZROWS, HHID), jnp.float32),
        pltpu.VMEM_SHARED((NPAD, HHID), jnp.float32),
        pltpu.SemaphoreType.DMA,
        pltpu.SemaphoreType.DMA,
        pltpu.SemaphoreType.DMA,
        pltpu.SemaphoreType.DMA,
        pltpu.SemaphoreType.DMA,
        pltpu.SemaphoreType.DMA,
        pltpu.SemaphoreType.DMA,
        pltpu.SemaphoreType.DMA,
    ],
    compiler_params=_sc_params,
)
def _edge_kernel(g_hbm, src2_hbm, dstr_hbm, out_hbm, sbuf, dbuf, rows,
                 zb_v, acc_sh, si0, si1, sj0, sj1, sg0, sg1, ss0, ss1):
    c = lax.axis_index("c")
    s = lax.axis_index("s")
    sem_i = (si0, si1)
    sem_j = (sj0, sj1)
    sem_g = (sg0, sg1)
    sem_s = (ss0, ss1)

    zf = jnp.zeros((LANES,), jnp.float32)

    def zfill(i, _):
        zb_v[i, :] = zf
        return 0
    lax.fori_loop(0, ZROWS, zfill, 0)
    row0 = s * TROWS
    for t in range(TROWS // ZROWS):
        pltpu.sync_copy(zb_v, acc_sh.at[pl.ds(row0 + t * ZROWS, ZROWS)])
    plsc.subcore_barrier()

    q, r = NBLK // NS, NBLK % NS
    cnt = q + jnp.where(s < r, 1, 0)
    bstart = s * q + jnp.minimum(s, r)
    cE = c * E

    def idx_descs(b, u):
        off = pl.multiple_of((bstart + b) * BLK, 8)
        d_i = pltpu.make_async_copy(
            src2_hbm.at[pl.ds(cE + off, BLK)], sbuf.at[u], sem_i[u])
        d_j = pltpu.make_async_copy(
            dstr_hbm.at[pl.ds((bstart + b) * K, K)], dbuf.at[u], sem_j[u])
        return d_i, d_j

    def gather_desc(u, k):
        return pltpu.make_async_copy(
            g_hbm.at[sbuf.at[u, pl.ds(k * CH, CH)]],
            rows.at[u, pl.ds(k * CH, CH)], sem_g[u])

    def scatter_desc(u, k):
        return pltpu.make_async_copy(
            rows.at[u, pl.ds(k * CH, CH)], acc_sh.at[dbuf.at[u, k]],
            sem_s[u])

    def stage(b, u):
        w = b - 1

        @pl.when((w >= 0) & (w < cnt))
        def _():
            d_i, d_j = idx_descs(w, 1 - u)
            d_i.wait()
            d_j.wait()
            for k in range(K):
                gather_desc(1 - u, k).start()

        v = b - 2

        @pl.when((v >= 0) & (v < cnt))
        def _():
            for k in range(K):
                gather_desc(u, k).wait()
            for k in range(K):
                pltpu.async_copy(rows.at[u, pl.ds(k * CH, CH)],
                                 acc_sh.at[dbuf.at[u, k]], sem_s[u],
                                 add=True)
            for k in range(K):
                scatter_desc(u, k).wait()

        @pl.when(b < cnt)
        def _():
            d_i, d_j = idx_descs(b, u)
            d_i.start()
            d_j.start()

    def body(js, _):
        stage(2 * js, 0)
        stage(2 * js + 1, 1)
        return 0
    lax.fori_loop(0, (cnt + 3) // 2, body, 0)

    plsc.subcore_barrier()
    for t in range(TROWS // ZROWS):
        pltpu.sync_copy(acc_sh.at[pl.ds(row0 + t * ZROWS, ZROWS)], zb_v)
        pltpu.sync_copy(zb_v, out_hbm.at[c, pl.ds(row0 + t * ZROWS, ZROWS)])


def _dense1_body(x_ref, dp_ref, w1_ref, dinv_ref, h1_ref, g_ref):
    deg = dp_ref[0] + dp_ref[1] + 1.0
    dinv = lax.rsqrt(deg)[:, None]
    h1 = jnp.dot(x_ref[...], w1_ref[...], preferred_element_type=jnp.float32)
    g = dinv * h1
    dinv_ref[...] = dinv
    h1_ref[...] = h1
    g_ref[...] = jnp.stack([g[:, :HHID], g[:, HHID:]], axis=0)


def _dense1(x, dp, W1):
    return pl.pallas_call(
        _dense1_body,
        grid=(GRID,),
        in_specs=[
            pl.BlockSpec((RB, 11), lambda i: (i, 0)),
            pl.BlockSpec((NC, RB), lambda i: (0, i)),
            pl.BlockSpec((11, HID), lambda i: (0, 0)),
        ],
        out_specs=[
            pl.BlockSpec((RB, 1), lambda i: (i, 0)),
            pl.BlockSpec((RB, HID), lambda i: (i, 0)),
            pl.BlockSpec((NC, RB, HHID), lambda i: (0, i, 0)),
        ],
        out_shape=[
            jax.ShapeDtypeStruct((NPAD, 1), jnp.float32),
            jax.ShapeDtypeStruct((NPAD, HID), jnp.float32),
            jax.ShapeDtypeStruct((NC, NPAD, HHID), jnp.float32),
        ],
    )(x, dp, W1)


def _dense2_body(acc_ref, dinv_ref, h1_ref, w2_ref, b1_ref, h2_ref, g_ref):
    dinv = dinv_ref[...]
    agg = jnp.concatenate([acc_ref[0], acc_ref[1]], axis=-1)
    pre = dinv * agg + (dinv * dinv) * h1_ref[...] + b1_ref[...]
    h = jnp.maximum(pre, 0.0)
    h2 = jnp.dot(h, w2_ref[...], preferred_element_type=jnp.float32)
    g2 = dinv * h2
    h2_ref[...] = h2
    g_ref[...] = jnp.stack([g2[:, :HHID], g2[:, HHID:]], axis=0)


def _dense2(acc, dinv, h1, W2, b1):
    return pl.pallas_call(
        _dense2_body,
        grid=(GRID,),
        in_specs=[
            pl.BlockSpec((NC, RB, HHID), lambda i: (0, i, 0)),
            pl.BlockSpec((RB, 1), lambda i: (i, 0)),
            pl.BlockSpec((RB, HID), lambda i: (i, 0)),
            pl.BlockSpec((HID, HID), lambda i: (0, 0)),
            pl.BlockSpec((1, HID), lambda i: (0, 0)),
        ],
        out_specs=[
            pl.BlockSpec((RB, HID), lambda i: (i, 0)),
            pl.BlockSpec((NC, RB, HHID), lambda i: (0, i, 0)),
        ],
        out_shape=[
            jax.ShapeDtypeStruct((NPAD, HID), jnp.float32),
            jax.ShapeDtypeStruct((NC, NPAD, HHID), jnp.float32),
        ],
    )(acc, dinv, h1, W2, b1)


def _dense3_body(acc_ref, dinv_ref, h2_ref, b2_ref, wo1_ref, bo1_ref,
                 wo2_ref, bo2_ref, y_ref):
    dinv = dinv_ref[...]
    agg = jnp.concatenate([acc_ref[0], acc_ref[1]], axis=-1)
    pre = dinv * agg + (dinv * dinv) * h2_ref[...] + b2_ref[...]
    h = jnp.maximum(pre, 0.0)
    t = jnp.dot(h, wo1_ref[...], preferred_element_type=jnp.float32)
    t = t + bo1_ref[...]
    t = jnp.where(t > 0, t, jnp.exp(t) - 1.0)
    y = jnp.dot(t, wo2_ref[...], preferred_element_type=jnp.float32)
    y_ref[...] = y + bo2_ref[...]


def _dense3(acc, dinv, h2, b2, Wo1, bo1, Wo2, bo2):
    return pl.pallas_call(
        _dense3_body,
        grid=(GRID,),
        in_specs=[
            pl.BlockSpec((NC, RB, HHID), lambda i: (0, i, 0)),
            pl.BlockSpec((RB, 1), lambda i: (i, 0)),
            pl.BlockSpec((RB, HID), lambda i: (i, 0)),
            pl.BlockSpec((1, HID), lambda i: (0, 0)),
            pl.BlockSpec((HID, HHID), lambda i: (0, 0)),
            pl.BlockSpec((1, HHID), lambda i: (0, 0)),
            pl.BlockSpec((HHID, 1), lambda i: (0, 0)),
            pl.BlockSpec((1, 1), lambda i: (0, 0)),
        ],
        out_specs=[pl.BlockSpec((RB, 1), lambda i: (i, 0))],
        out_shape=[jax.ShapeDtypeStruct((NPAD, 1), jnp.float32)],
    )(acc, dinv, h2, b2, Wo1, bo1, Wo2, bo2)


def kernel(x, edge_index, batch, W1, b1, W2, b2, Wo1, bo1, Wo2, bo2):
    del batch
    src = edge_index[0]
    dst = edge_index[1]
    src2 = jnp.concatenate([src, src + NPAD])
    dstr = dst.reshape(E // CH, CH)
    x_pad = jnp.concatenate(
        [x, jnp.zeros((NPAD - N, x.shape[1]), x.dtype)], axis=0)

    dp = _deg_kernel(dstr)
    dinv, h1, g1 = _dense1(x_pad, dp, W1)
    acc1 = _edge_kernel(g1.reshape(NC * NPAD, HHID), src2, dstr)
    h2, g2 = _dense2(acc1, dinv, h1, W2, b1.reshape(1, HID))
    acc2 = _edge_kernel(g2.reshape(NC * NPAD, HHID), src2, dstr)
    (y,) = _dense3(acc2, dinv, h2, b2.reshape(1, HID), Wo1,
                   bo1.reshape(1, HHID), Wo2, bo2.reshape(1, 1))
    return y[:N, 0]
